# Initial kernel scaffold; baseline (speedup 1.0000x reference)
import jax
import jax.numpy as jnp
from jax.experimental import pallas as pl


def kernel(h, adj, W_l, W_r, attn_w, attn_b):
    raise NotImplementedError("write your pallas kernel here")


# speedup vs baseline: 4.9517x; 4.9517x over previous
import jax
import jax.numpy as jnp
from jax.experimental import pallas as pl
from jax.experimental.pallas import tpu as pltpu

_H = 8
_HID = 32
_TOPM = 16
_NEG = float("-inf")


def _leaky(x):
    return jnp.maximum(x, 0.2 * x)


def _gat_step(gl_ref, gr_ref, grTp_ref, adjT_ref, wvec_ref, bias_ref,
              out_ref):
    N = gl_ref.shape[1]
    gl3 = gl_ref[0].reshape(N, _H, _HID)
    grTp3 = grTp_ref[0].reshape(_HID, _H, N)

    acc = jnp.zeros((N, _H, N), jnp.float32)
    for f in range(_HID):
        wv = wvec_ref[0, f]
        Lf = gl3[:, :, f:f + 1]
        Rf = grTp3[f]
        x = Lf + Rf[None, :, :]
        lkb = _leaky(x).astype(jnp.bfloat16).astype(jnp.float32)
        acc = acc + lkb * wv

    S = acc + bias_ref[0, 0]
    mask0 = (adjT_ref[0] == 0)[:, None, :]
    S = jnp.where(mask0, _NEG, S)

    m = S
    for _ in range(_TOPM - 1):
        rm = jnp.max(m, axis=0, keepdims=True)
        m = jnp.where(m >= rm, _NEG, m)
    thr = jnp.max(m, axis=0, keepdims=True)

    Sk = jnp.where(S >= thr, S, _NEG)
    mx = jnp.max(Sk, axis=0, keepdims=True)
    e = jnp.exp(Sk - mx)
    denom = jnp.sum(e, axis=0, keepdims=True)
    P = (e / denom).astype(jnp.bfloat16)
    grb = gr_ref[0].astype(jnp.bfloat16)

    for hh in range(_H):
        Ph = P[:, hh, :]
        gr_h = grb[:, hh * _HID:(hh + 1) * _HID]
        oh = jax.lax.dot_general(Ph, gr_h, (((0,), (0,)), ((), ())),
                                 preferred_element_type=jnp.float32)
        out_ref[0, :, hh * _HID:(hh + 1) * _HID] = _leaky(oh)


def kernel(h, adj, W_l, W_r, attn_w, attn_b):
    B, N, Fin = h.shape
    Fout = W_l.shape[0]
    gl = h @ W_l.T
    gr = h @ W_r.T
    o = jnp.arange(Fout)
    perm = (o % _H) * _HID + o // _H
    grTp = jnp.swapaxes(gr[:, :, perm], 1, 2)
    adjT = jnp.swapaxes(adj, 1, 2)
    wu = jax.lax.bitcast_convert_type(attn_w, jnp.uint32)
    wu = (wu + jnp.uint32(0x7FFF) + ((wu >> 16) & jnp.uint32(1))) & jnp.uint32(0xFFFF0000)
    wvec = jax.lax.bitcast_convert_type(wu, jnp.float32).reshape(1, _HID)
    bias = attn_b.reshape(1, 1)

    grid = (B,)
    out = pl.pallas_call(
        _gat_step,
        grid=grid,
        in_specs=[
            pl.BlockSpec((1, N, Fout), lambda b: (b, 0, 0)),
            pl.BlockSpec((1, N, Fout), lambda b: (b, 0, 0)),
            pl.BlockSpec((1, Fout, N), lambda b: (b, 0, 0)),
            pl.BlockSpec((1, N, N), lambda b: (b, 0, 0)),
            pl.BlockSpec((1, _HID), lambda b: (0, 0), memory_space=pltpu.SMEM),
            pl.BlockSpec((1, 1), lambda b: (0, 0), memory_space=pltpu.SMEM),
        ],
        out_specs=pl.BlockSpec((1, N, Fout), lambda b: (b, 0, 0)),
        out_shape=jax.ShapeDtypeStruct((B, N, Fout), jnp.float32),
    )(gl, gr, grTp, adjT, wvec, bias)
    return out


# speedup vs baseline: 5.0665x; 1.0232x over previous
import jax
import jax.numpy as jnp
from jax.experimental import pallas as pl
from jax.experimental.pallas import tpu as pltpu

_H = 8
_HID = 32
_TOPM = 16
_NEG = float("-inf")


def _leaky(x):
    return jnp.maximum(x, 0.2 * x)


def _gat_step(gl_ref, gr_ref, grTp_ref, adjT_ref, wvec_ref, bias_ref,
              out_ref):
    N = gl_ref.shape[1]
    gl3 = gl_ref[0].reshape(N, _H, _HID)
    grTp3 = grTp_ref[0].reshape(_HID, _H, N)

    acc = jnp.zeros((N, _H, N), jnp.float32)
    for f in range(_HID):
        wv = wvec_ref[0, f]
        Lf = gl3[:, :, f:f + 1]
        Rf = grTp3[f]
        x = Lf + Rf[None, :, :]
        lkb = _leaky(x).astype(jnp.bfloat16).astype(jnp.float32)
        acc = acc + lkb * wv

    S = acc + bias_ref[0, 0]
    adjT = adjT_ref[0].T
    mask0 = (adjT == 0)[:, None, :]
    S = jnp.where(mask0, _NEG, S)

    m = S
    mx = None
    for it in range(_TOPM - 1):
        rm = jnp.max(m, axis=0, keepdims=True)
        if it == 0:
            mx = rm
        m = jnp.where(m >= rm, _NEG, m)
    thr = jnp.max(m, axis=0, keepdims=True)

    Sk = jnp.where(S >= thr, S, _NEG)
    e = jnp.exp(Sk - mx)
    denom = jnp.sum(e, axis=0, keepdims=True)
    P = (e / denom).astype(jnp.bfloat16)
    grb = gr_ref[0].astype(jnp.bfloat16)

    for hh in range(_H):
        Ph = P[:, hh, :]
        gr_h = grb[:, hh * _HID:(hh + 1) * _HID]
        oh = jax.lax.dot_general(Ph, gr_h, (((0,), (0,)), ((), ())),
                                 preferred_element_type=jnp.float32)
        out_ref[0, :, hh * _HID:(hh + 1) * _HID] = _leaky(oh)


def kernel(h, adj, W_l, W_r, attn_w, attn_b):
    B, N, Fin = h.shape
    Fout = W_l.shape[0]
    gl = h @ W_l.T
    gr = h @ W_r.T
    o = jnp.arange(Fout)
    perm = (o % _H) * _HID + o // _H
    grTp = jnp.swapaxes(gr[:, :, perm], 1, 2)
    wu = jax.lax.bitcast_convert_type(attn_w, jnp.uint32)
    wu = (wu + jnp.uint32(0x7FFF) + ((wu >> 16) & jnp.uint32(1))) & jnp.uint32(0xFFFF0000)
    wvec = jax.lax.bitcast_convert_type(wu, jnp.float32).reshape(1, _HID)
    bias = attn_b.reshape(1, 1)

    grid = (B,)
    out = pl.pallas_call(
        _gat_step,
        grid=grid,
        in_specs=[
            pl.BlockSpec((1, N, Fout), lambda b: (b, 0, 0)),
            pl.BlockSpec((1, N, Fout), lambda b: (b, 0, 0)),
            pl.BlockSpec((1, Fout, N), lambda b: (b, 0, 0)),
            pl.BlockSpec((1, N, N), lambda b: (b, 0, 0)),
            pl.BlockSpec((1, _HID), lambda b: (0, 0), memory_space=pltpu.SMEM),
            pl.BlockSpec((1, 1), lambda b: (0, 0), memory_space=pltpu.SMEM),
        ],
        out_specs=pl.BlockSpec((1, N, Fout), lambda b: (b, 0, 0)),
        out_shape=jax.ShapeDtypeStruct((B, N, Fout), jnp.float32),
    )(gl, gr, grTp, adj, wvec, bias)
    return out
